# baseline (speedup 1.0000x reference)
import functools

import jax
import jax.numpy as jnp
from jax import lax
from jax.experimental import pallas as pl
from jax.experimental.pallas import tpu as pltpu
from jax.experimental.pallas import tpu_sc as plsc

_B = 4096
_D = 64
_L = 16


@jax.jit
def _gmf(user_ids, item_ids, user_table, item_table):
    info = plsc.get_sparse_core_info()
    nc, ns = info.num_cores, info.num_subcores
    nw = nc * ns
    b_per_w = _B // nw

    mesh = plsc.VectorSubcoreMesh(core_axis_name="c", subcore_axis_name="s")

    @functools.partial(
        pl.kernel,
        mesh=mesh,
        out_type=jax.ShapeDtypeStruct((_B, _D), jnp.float32),
        scratch_types=[
            pltpu.VMEM((b_per_w,), jnp.int32),
            pltpu.VMEM((b_per_w,), jnp.int32),
            pltpu.VMEM((b_per_w, _D), jnp.float32),
            pltpu.VMEM((b_per_w, _D), jnp.float32),
            pltpu.SemaphoreType.DMA,
        ],
        compiler_params=pltpu.CompilerParams(use_tc_tiling_on_sc=False),
    )
    def k(uid_hbm, iid_hbm, utab_hbm, itab_hbm, out_hbm,
          uidx_v, iidx_v, urows_v, irows_v, sem):
        wid = lax.axis_index("s") * nc + lax.axis_index("c")
        base = wid * b_per_w
        pltpu.sync_copy(uid_hbm.at[pl.ds(base, b_per_w)], uidx_v)
        cu = pltpu.async_copy(utab_hbm.at[uidx_v], urows_v, sem)
        pltpu.sync_copy(iid_hbm.at[pl.ds(base, b_per_w)], iidx_v)
        ci = pltpu.async_copy(itab_hbm.at[iidx_v], irows_v, sem)
        cu.wait()
        ci.wait()

        def body(r, carry):
            for c in range(_D // _L):
                s = pl.ds(c * _L, _L)
                urows_v[r, s] = urows_v[r, s] * irows_v[r, s]
            return carry

        lax.fori_loop(0, b_per_w, body, 0)
        pltpu.sync_copy(urows_v, out_hbm.at[pl.ds(base, b_per_w)])

    return k(user_ids, item_ids, user_table, item_table)


def kernel(user_ids, item_ids, user_table, item_table):
    return _gmf(user_ids, item_ids, user_table, item_table)


# speedup vs baseline: 1.4273x; 1.4273x over previous
import functools

import jax
import jax.numpy as jnp
from jax import lax
from jax.experimental import pallas as pl
from jax.experimental.pallas import tpu as pltpu
from jax.experimental.pallas import tpu_sc as plsc

_B = 4096
_D = 64
_L = 16


@jax.jit
def _gmf(user_ids, item_ids, user_table, item_table):
    info = plsc.get_sparse_core_info()
    nc, ns = info.num_cores, info.num_subcores
    nw = nc * ns
    b_per_w = _B // nw

    mesh = plsc.VectorSubcoreMesh(core_axis_name="c", subcore_axis_name="s")

    @functools.partial(
        pl.kernel,
        mesh=mesh,
        out_type=jax.ShapeDtypeStruct((_B, _D), jnp.float32),
        scratch_types=[
            pltpu.VMEM((b_per_w,), jnp.int32),
            pltpu.VMEM((b_per_w,), jnp.int32),
            pltpu.VMEM((b_per_w, _D), jnp.float32),
            pltpu.VMEM((b_per_w, _D), jnp.float32),
            pltpu.SemaphoreType.DMA,
            pltpu.SemaphoreType.DMA,
        ],
        compiler_params=pltpu.CompilerParams(needs_layout_passes=False),
    )
    def k(uid_hbm, iid_hbm, utab_hbm, itab_hbm, out_hbm,
          uidx_v, iidx_v, urows_v, irows_v, semu, semi):
        wid = lax.axis_index("s") * nc + lax.axis_index("c")
        base = wid * b_per_w
        pltpu.sync_copy(uid_hbm.at[pl.ds(base, b_per_w)], uidx_v)
        pltpu.sync_copy(iid_hbm.at[pl.ds(base, b_per_w)], iidx_v)

        lanes = lax.iota(jnp.int32, 16)

        def fire(cidx, carry):
            cbase = cidx * _L
            uvec = uidx_v[pl.ds(cbase, _L)]
            ivec = iidx_v[pl.ds(cbase, _L)]
            for l in range(_L):
                ur = jnp.sum(jnp.where(lanes == l, uvec, 0))
                ir = jnp.sum(jnp.where(lanes == l, ivec, 0))
                r = cbase + l
                pltpu.async_copy(utab_hbm.at[pl.ds(ur, 1), :],
                                 urows_v.at[pl.ds(r, 1), :], semu)
                pltpu.async_copy(itab_hbm.at[pl.ds(ir, 1), :],
                                 irows_v.at[pl.ds(r, 1), :], semi)
            return carry

        lax.fori_loop(0, b_per_w // _L, fire, 0)
        pltpu.make_async_copy(utab_hbm.at[pl.ds(0, b_per_w), :], urows_v,
                              semu).wait()
        pltpu.make_async_copy(itab_hbm.at[pl.ds(0, b_per_w), :], irows_v,
                              semi).wait()

        def body(r, carry):
            for c in range(_D // _L):
                s = pl.ds(c * _L, _L)
                urows_v[r, s] = urows_v[r, s] * irows_v[r, s]
            return carry

        lax.fori_loop(0, b_per_w, body, 0)
        pltpu.sync_copy(urows_v, out_hbm.at[pl.ds(base, b_per_w)])

    return k(user_ids, item_ids, user_table, item_table)


def kernel(user_ids, item_ids, user_table, item_table):
    return _gmf(user_ids, item_ids, user_table, item_table)


# speedup vs baseline: 1.4318x; 1.0032x over previous
import functools

import jax
import jax.numpy as jnp
from jax import lax
from jax.experimental import pallas as pl
from jax.experimental.pallas import tpu as pltpu
from jax.experimental.pallas import tpu_sc as plsc

_B = 4096
_D = 64
_L = 16


@jax.jit
def _gmf(user_ids, item_ids, user_table, item_table):
    info = plsc.get_sparse_core_info()
    nc, ns = info.num_cores, info.num_subcores
    nw = nc * ns
    b_per_w = _B // nw

    mesh = plsc.VectorSubcoreMesh(core_axis_name="c", subcore_axis_name="s")

    @functools.partial(
        pl.kernel,
        mesh=mesh,
        out_type=jax.ShapeDtypeStruct((_B, _D), jnp.float32),
        scratch_types=[
            pltpu.VMEM((b_per_w,), jnp.int32),
            pltpu.VMEM((b_per_w,), jnp.int32),
            pltpu.VMEM((b_per_w, _D), jnp.float32),
            pltpu.VMEM((b_per_w, _D), jnp.float32),
            pltpu.SemaphoreType.DMA,
            pltpu.SemaphoreType.DMA,
        ],
        compiler_params=pltpu.CompilerParams(needs_layout_passes=False,
                                             skip_device_barrier=True),
    )
    def k(uid_hbm, iid_hbm, utab_hbm, itab_hbm, out_hbm,
          uidx_v, iidx_v, urows_v, irows_v, semu, semi):
        wid = lax.axis_index("s") * nc + lax.axis_index("c")
        base = wid * b_per_w
        pltpu.sync_copy(uid_hbm.at[pl.ds(base, b_per_w)], uidx_v)
        pltpu.sync_copy(iid_hbm.at[pl.ds(base, b_per_w)], iidx_v)

        lanes = lax.iota(jnp.int32, 16)

        def fire(cidx, carry):
            cbase = cidx * _L
            uvec = uidx_v[pl.ds(cbase, _L)]
            ivec = iidx_v[pl.ds(cbase, _L)]
            for l in range(_L):
                ur = jnp.sum(jnp.where(lanes == l, uvec, 0))
                ir = jnp.sum(jnp.where(lanes == l, ivec, 0))
                r = cbase + l
                pltpu.async_copy(utab_hbm.at[pl.ds(ur, 1), :],
                                 urows_v.at[pl.ds(r, 1), :], semu)
                pltpu.async_copy(itab_hbm.at[pl.ds(ir, 1), :],
                                 irows_v.at[pl.ds(r, 1), :], semi)
            return carry

        lax.fori_loop(0, b_per_w // _L, fire, 0)
        pltpu.make_async_copy(utab_hbm.at[pl.ds(0, b_per_w), :], urows_v,
                              semu).wait()
        pltpu.make_async_copy(itab_hbm.at[pl.ds(0, b_per_w), :], irows_v,
                              semi).wait()

        def body(r, carry):
            for c in range(_D // _L):
                s = pl.ds(c * _L, _L)
                urows_v[r, s] = urows_v[r, s] * irows_v[r, s]
            return carry

        lax.fori_loop(0, b_per_w, body, 0)
        pltpu.sync_copy(urows_v, out_hbm.at[pl.ds(base, b_per_w)])

    return k(user_ids, item_ids, user_table, item_table)


def kernel(user_ids, item_ids, user_table, item_table):
    return _gmf(user_ids, item_ids, user_table, item_table)
